# Initial kernel scaffold; baseline (speedup 1.0000x reference)
import jax
import jax.numpy as jnp
from jax.experimental import pallas as pl


def kernel(x, edge_attr, edge_index, edge_type, batch, W0, root0, b0, g0, be0, W1, root1, b1, g1, be1, W2, root2, b2, g2, be2, W3, root3, b3, g3, be3, lin1_W, lin1_b, lin2_W, lin2_b):
    raise NotImplementedError("write your pallas kernel here")


# speedup vs baseline: 4.9477x; 4.9477x over previous
import functools

import jax
import jax.numpy as jnp
from jax import lax
from jax.experimental import pallas as pl
from jax.experimental.pallas import tpu as pltpu
from jax.experimental.pallas import tpu_sc as plsc

N = 10000
NG = 64
R = 3
F = 128
FQ = 32
NT = 16
NC = 2
K = 128
NSTEP = 160
EPAD = NT * NSTEP * K
NSTEP32 = EPAD // (NC * NT * K)
NROW = 30080
DUMMY = R * N
RPT = NROW // NT


def _sc_aggregate_body(table, gidxa, gidxb, sidx, zrows, out, gv, sv, rows,
                       acc, sem):
    c = lax.axis_index("c")
    s = lax.axis_index("s")
    w = c * NT + s
    pltpu.sync_copy(sidx.at[s], sv)

    def step(j, carry):
        pltpu.async_copy(table.at[gv.at[j]], rows, sem).wait()
        pltpu.sync_copy(rows, acc.at[sv.at[j]], add=True)
        return carry

    for p, gsrc in enumerate((gidxa, gidxb)):
        pltpu.sync_copy(zrows, acc.at[pl.ds(s * RPT, RPT)])
        pltpu.sync_copy(gsrc.at[w], gv)
        plsc.subcore_barrier()
        lax.fori_loop(0, NSTEP, step, 0)
        plsc.subcore_barrier()
        pltpu.sync_copy(acc.at[pl.ds(s * RPT, RPT)],
                        out.at[c].at[p].at[pl.ds(s * RPT, RPT)])


def _sc_count_body(sidx32, ones_in, zrows, out, sv, ones_v, acc):
    c = lax.axis_index("c")
    s = lax.axis_index("s")
    w = c * NT + s
    pltpu.sync_copy(zrows, acc.at[pl.ds(s * RPT, RPT)])
    pltpu.sync_copy(sidx32.at[w], sv)
    pltpu.sync_copy(ones_in, ones_v)
    plsc.subcore_barrier()

    def step(j, carry):
        pltpu.sync_copy(ones_v, acc.at[sv.at[j]], add=True)
        return carry

    lax.fori_loop(0, NSTEP32, step, 0)
    plsc.subcore_barrier()
    pltpu.sync_copy(acc.at[pl.ds(s * RPT, RPT)],
                    out.at[c].at[pl.ds(s * RPT, RPT)])


@functools.cache
def _sc_kernels():
    mesh = plsc.VectorSubcoreMesh(core_axis_name="c", subcore_axis_name="s")
    params = pltpu.CompilerParams(use_tc_tiling_on_sc=False)
    agg = pl.kernel(
        _sc_aggregate_body,
        out_type=jax.ShapeDtypeStruct((NC, 2, NROW, FQ), jnp.float32),
        mesh=mesh,
        compiler_params=params,
        scratch_types=[
            pltpu.VMEM((NSTEP, K), jnp.int32),
            pltpu.VMEM((NSTEP, K), jnp.int32),
            pltpu.VMEM((K, FQ), jnp.float32),
            pltpu.VMEM_SHARED((NROW, FQ), jnp.float32),
            pltpu.SemaphoreType.DMA,
        ],
    )
    cnt = pl.kernel(
        _sc_count_body,
        out_type=jax.ShapeDtypeStruct((NC, NROW, 8), jnp.float32),
        mesh=mesh,
        compiler_params=params,
        scratch_types=[
            pltpu.VMEM((NSTEP32, K), jnp.int32),
            pltpu.VMEM((K, 8), jnp.float32),
            pltpu.VMEM_SHARED((NROW, 8), jnp.float32),
        ],
    )
    return agg, cnt


B = 2000
NB = N // B


def _tc_accum_body(h_ref, p0_ref, p1_ref, p2_ref, c0_ref, c1_ref, c2_ref,
                   root_ref, b_ref, W_ref, t_ref, stats_ref, ssum, ssq):
    i = pl.program_id(0)
    t = jnp.dot(h_ref[...], root_ref[...],
                preferred_element_type=jnp.float32) + b_ref[...]
    for r, (p_ref, c_ref) in enumerate(
            ((p0_ref, c0_ref), (p1_ref, c1_ref), (p2_ref, c2_ref))):
        cnt = c_ref[0, :, 0:1] + c_ref[1, :, 0:1]
        inv = 1.0 / jnp.maximum(cnt, 1.0)
        for q in range(4):
            c, p = q % 2, q // 2
            t = t + jnp.dot(p_ref[c, p, :, :] * inv,
                            W_ref[r, q * FQ:(q + 1) * FQ, :],
                            preferred_element_type=jnp.float32)
    t_ref[...] = t
    ps = jnp.sum(t, axis=0, keepdims=True)
    pq = jnp.sum(t * t, axis=0, keepdims=True)

    @pl.when(i == 0)
    def _():
        ssum[...] = ps
        ssq[...] = pq

    @pl.when(i > 0)
    def _():
        ssum[...] += ps
        ssq[...] += pq

    @pl.when(i == NB - 1)
    def _():
        mu = ssum[...] / N
        stats_ref[0:1, :] = mu
        stats_ref[1:2, :] = ssq[...] / N - mu * mu


def _part_specs():
    specs = [pl.BlockSpec((B, F), lambda i: (i, 0))]
    for r in range(R):
        specs.append(pl.BlockSpec((NC, 2, B, FQ),
                                  lambda i, r=r: (0, 0, r * NB + i, 0)))
    for r in range(R):
        specs.append(pl.BlockSpec((NC, B, 8),
                                  lambda i, r=r: (0, r * NB + i, 0)))
    specs += [pl.BlockSpec((F, F), lambda i: (0, 0)),
              pl.BlockSpec((1, F), lambda i: (0, 0)),
              pl.BlockSpec((R, F, F), lambda i: (0, 0, 0))]
    return specs


_tc_accum = pl.pallas_call(
    _tc_accum_body,
    grid=(NB,),
    in_specs=_part_specs(),
    out_specs=[pl.BlockSpec((B, F), lambda i: (i, 0)),
               pl.BlockSpec((8, F), lambda i: (0, 0))],
    out_shape=[jax.ShapeDtypeStruct((N, F), jnp.float32),
               jax.ShapeDtypeStruct((8, F), jnp.float32)],
    scratch_shapes=[pltpu.VMEM((1, F), jnp.float32),
                    pltpu.VMEM((1, F), jnp.float32)],
)


def _normed(t_ref, stats_ref, g_ref, be_ref):
    mu = stats_ref[0:1, :]
    var = stats_ref[1:2, :]
    return ((t_ref[...] - mu) * lax.rsqrt(var + 1e-5) * g_ref[...]
            + be_ref[...])


def _tc_norm_body(t_ref, stats_ref, g_ref, be_ref, hout_ref, hsplit_ref):
    tn = _normed(t_ref, stats_ref, g_ref, be_ref)
    hout_ref[...] = tn
    for q in range(4):
        hsplit_ref[q, :, :] = tn[:, q * FQ:(q + 1) * FQ]


_tc_norm = pl.pallas_call(
    _tc_norm_body,
    grid=(NB,),
    in_specs=[pl.BlockSpec((B, F), lambda i: (i, 0)),
              pl.BlockSpec((8, F), lambda i: (0, 0)),
              pl.BlockSpec((1, F), lambda i: (0, 0)),
              pl.BlockSpec((1, F), lambda i: (0, 0))],
    out_specs=[pl.BlockSpec((B, F), lambda i: (i, 0)),
               pl.BlockSpec((4, B, FQ), lambda i: (0, i, 0))],
    out_shape=[jax.ShapeDtypeStruct((N, F), jnp.float32),
               jax.ShapeDtypeStruct((4, N, FQ), jnp.float32)],
)


def _tc_final_body(t_ref, stats_ref, g_ref, be_ref, batch_ref, l1w_ref,
                   l1b_ref, l2w_ref, l2b_ref, out_ref, seg, cg):
    i = pl.program_id(0)
    tn = _normed(t_ref, stats_ref, g_ref, be_ref)
    gi = lax.broadcasted_iota(jnp.int32, (B, NG), 1)
    oh = (gi == batch_ref[...]).astype(jnp.float32)
    dn = (((0,), (0,)), ((), ()))
    segp = lax.dot_general(oh, tn, dn,
                           preferred_element_type=jnp.float32)
    cgp = lax.dot_general(oh, jnp.ones((B, 8), jnp.float32), dn,
                          preferred_element_type=jnp.float32)

    @pl.when(i == 0)
    def _():
        seg[...] = segp
        cg[...] = cgp

    @pl.when(i > 0)
    def _():
        seg[...] += segp
        cg[...] += cgp

    @pl.when(i == NB - 1)
    def _():
        pooled = seg[...] / jnp.maximum(cg[..., 0:1], 1.0)
        y = jnp.dot(pooled, l1w_ref[...],
                    preferred_element_type=jnp.float32) + l1b_ref[...]
        out_ref[...] = jnp.dot(y, l2w_ref[...],
                               preferred_element_type=jnp.float32) + l2b_ref[...]


_tc_final = pl.pallas_call(
    _tc_final_body,
    grid=(NB,),
    in_specs=[pl.BlockSpec((B, F), lambda i: (i, 0)),
              pl.BlockSpec((8, F), lambda i: (0, 0)),
              pl.BlockSpec((1, F), lambda i: (0, 0)),
              pl.BlockSpec((1, F), lambda i: (0, 0)),
              pl.BlockSpec((B, 1), lambda i: (i, 0)),
              pl.BlockSpec((F, F), lambda i: (0, 0)),
              pl.BlockSpec((1, F), lambda i: (0, 0)),
              pl.BlockSpec((F, F), lambda i: (0, 0)),
              pl.BlockSpec((1, F), lambda i: (0, 0))],
    out_specs=pl.BlockSpec((NG, F), lambda i: (0, 0)),
    out_shape=jax.ShapeDtypeStruct((NG, F), jnp.float32),
    scratch_shapes=[pltpu.VMEM((NG, F), jnp.float32),
                    pltpu.VMEM((NG, 8), jnp.float32)],
)


def _pad_layer(Wl, rootl, bl, gl, bel):
    fi, fo = rootl.shape
    Wl = jnp.pad(Wl, ((0, 0), (0, F - fi), (0, F - fo)))
    rootl = jnp.pad(rootl, ((0, F - fi), (0, F - fo)))
    pad1 = lambda v: jnp.pad(v, (0, F - v.shape[0])).reshape(1, F)
    return Wl, rootl, pad1(bl), pad1(gl), pad1(bel)


def kernel(x, edge_attr, edge_index, edge_type, batch, W0, root0, b0, g0,
           be0, W1, root1, b1, g1, be1, W2, root2, b2, g2, be2, W3, root3,
           b3, g3, be3, lin1_W, lin1_b, lin2_W, lin2_b):
    src = edge_index[0].astype(jnp.int32)
    dst = edge_index[1].astype(jnp.int32)
    et = edge_type.astype(jnp.int32)
    E = src.shape[0]

    src_p = jnp.pad(src, (0, EPAD - E))
    sidx_flat = jnp.pad(et * N + dst, (0, EPAD - E), constant_values=DUMMY)
    gidx16 = src_p.reshape(NT, NSTEP, K)
    gidxa = jnp.concatenate([gidx16[None], gidx16[None] + N], axis=0)
    gidxa = gidxa.reshape(NC * NT, NSTEP, K)
    gidxb = gidxa + 2 * N
    sidx = sidx_flat.reshape(NT, NSTEP, K)
    sidx32 = sidx_flat.reshape(NC * NT, NSTEP32, K)

    zrows = jnp.zeros((RPT, FQ), jnp.float32)
    zrows8 = jnp.zeros((RPT, 8), jnp.float32)
    ones_in = jnp.ones((K, 8), jnp.float32)

    _sc_aggregate, _sc_count = _sc_kernels()
    cpart = _sc_count(sidx32, ones_in, zrows8)

    lw = [_pad_layer(W0, root0, b0, g0, be0),
          _pad_layer(W1, root1, b1, g1, be1),
          _pad_layer(W2, root2, b2, g2, be2),
          _pad_layer(W3, root3, b3, g3, be3)]

    h = x
    hsplit = jnp.concatenate([x[:, q * FQ:(q + 1) * FQ] for q in range(4)],
                             axis=0)
    for l in range(3):
        Wl, rootl, bl, gl, bel = lw[l]
        part = _sc_aggregate(hsplit, gidxa, gidxb, sidx, zrows)
        t, stats = _tc_accum(h, part, part, part, cpart, cpart, cpart,
                             rootl, bl, Wl)
        h, hsplit4 = _tc_norm(t, stats, gl, bel)
        hsplit = hsplit4.reshape(4 * N, FQ)

    Wl, rootl, bl, gl, bel = lw[3]
    part = _sc_aggregate(hsplit, gidxa, gidxb, sidx, zrows)
    t, stats = _tc_accum(h, part, part, part, cpart, cpart, cpart,
                         rootl, bl, Wl)
    l1w = jnp.pad(lin1_W, ((0, 0), (0, F - lin1_W.shape[1])))
    l1b = jnp.pad(lin1_b, (0, F - lin1_b.shape[0])).reshape(1, F)
    l2w = jnp.pad(lin2_W, ((0, F - lin2_W.shape[0]), (0, F - lin2_W.shape[1])))
    l2b = jnp.pad(lin2_b, (0, F - lin2_b.shape[0])).reshape(1, F)
    out = _tc_final(t, stats, gl, bel,
                    batch.astype(jnp.int32).reshape(N, 1), l1w, l1b, l2w,
                    l2b)
    return out[:, 0:1]


# speedup vs baseline: 6.7787x; 1.3701x over previous
import functools

import jax
import jax.numpy as jnp
from jax import lax
from jax.experimental import pallas as pl
from jax.experimental.pallas import tpu as pltpu
from jax.experimental.pallas import tpu_sc as plsc

N = 10000
NG = 64
R = 3
F = 128
FQ = 32
NT = 16
NC = 2
K = 128
NSTEP = 160
EPAD = NT * NSTEP * K
NSTEP32 = EPAD // (NC * NT * K)
NROW = 30080
DUMMY = R * N
RPT = NROW // NT


NBUF = 4


def _sc_aggregate_body(table, gidxa, gidxb, sidx, zrows, out, gv, sv, rows,
                       acc, *sems):
    gsem = sems[:NBUF]
    ssem = sems[NBUF:]
    c = lax.axis_index("c")
    s = lax.axis_index("s")
    w = c * NT + s
    pltpu.sync_copy(sidx.at[s], sv)

    def rnd(j0, carry):
        base = j0 * NBUF
        for b in range(NBUF):
            pltpu.make_async_copy(table.at[gv.at[base + b]], rows.at[b],
                                  gsem[b]).wait()
            pltpu.async_copy(rows.at[b], acc.at[sv.at[base + b]], ssem[b],
                             add=True)
        for b in range(NBUF):
            pltpu.make_async_copy(rows.at[b], acc.at[sv.at[base + b]],
                                  ssem[b]).wait()

            @pl.when(base + NBUF + b < NSTEP)
            def _():
                pltpu.async_copy(table.at[gv.at[base + NBUF + b]],
                                 rows.at[b], gsem[b])
        return carry

    for p, gsrc in enumerate((gidxa, gidxb)):
        pltpu.sync_copy(zrows, acc.at[pl.ds(s * RPT, RPT)])
        pltpu.sync_copy(gsrc.at[w], gv)
        plsc.subcore_barrier()
        for b in range(NBUF):
            pltpu.async_copy(table.at[gv.at[b]], rows.at[b], gsem[b])
        lax.fori_loop(0, NSTEP // NBUF, rnd, 0)
        plsc.subcore_barrier()
        pltpu.sync_copy(acc.at[pl.ds(s * RPT, RPT)],
                        out.at[c].at[p].at[pl.ds(s * RPT, RPT)])


def _sc_count_body(sidx32, ones_in, zrows, out, sv, ones_v, acc):
    c = lax.axis_index("c")
    s = lax.axis_index("s")
    w = c * NT + s
    pltpu.sync_copy(zrows, acc.at[pl.ds(s * RPT, RPT)])
    pltpu.sync_copy(sidx32.at[w], sv)
    pltpu.sync_copy(ones_in, ones_v)
    plsc.subcore_barrier()

    def step(j, carry):
        pltpu.sync_copy(ones_v, acc.at[sv.at[j]], add=True)
        return carry

    lax.fori_loop(0, NSTEP32, step, 0)
    plsc.subcore_barrier()
    pltpu.sync_copy(acc.at[pl.ds(s * RPT, RPT)],
                    out.at[c].at[pl.ds(s * RPT, RPT)])


@functools.cache
def _sc_kernels():
    mesh = plsc.VectorSubcoreMesh(core_axis_name="c", subcore_axis_name="s")
    params = pltpu.CompilerParams(use_tc_tiling_on_sc=False)
    agg = pl.kernel(
        _sc_aggregate_body,
        out_type=jax.ShapeDtypeStruct((NC, 2, NROW, FQ), jnp.float32),
        mesh=mesh,
        compiler_params=params,
        scratch_types=[
            pltpu.VMEM((NSTEP, K), jnp.int32),
            pltpu.VMEM((NSTEP, K), jnp.int32),
            pltpu.VMEM((NBUF, K, FQ), jnp.float32),
            pltpu.VMEM_SHARED((NROW, FQ), jnp.float32),
        ] + [pltpu.SemaphoreType.DMA] * (2 * NBUF),
    )
    cnt = pl.kernel(
        _sc_count_body,
        out_type=jax.ShapeDtypeStruct((NC, NROW, 8), jnp.float32),
        mesh=mesh,
        compiler_params=params,
        scratch_types=[
            pltpu.VMEM((NSTEP32, K), jnp.int32),
            pltpu.VMEM((K, 8), jnp.float32),
            pltpu.VMEM_SHARED((NROW, 8), jnp.float32),
        ],
    )
    return agg, cnt


B = 2000
NB = N // B


def _tc_accum_body(h_ref, p0_ref, p1_ref, p2_ref, c0_ref, c1_ref, c2_ref,
                   root_ref, b_ref, W_ref, t_ref, stats_ref, ssum, ssq):
    i = pl.program_id(0)
    t = jnp.dot(h_ref[...], root_ref[...],
                preferred_element_type=jnp.float32) + b_ref[...]
    for r, (p_ref, c_ref) in enumerate(
            ((p0_ref, c0_ref), (p1_ref, c1_ref), (p2_ref, c2_ref))):
        cnt = c_ref[0, :, 0:1] + c_ref[1, :, 0:1]
        inv = 1.0 / jnp.maximum(cnt, 1.0)
        for q in range(4):
            c, p = q % 2, q // 2
            t = t + jnp.dot(p_ref[c, p, :, :] * inv,
                            W_ref[r, q * FQ:(q + 1) * FQ, :],
                            preferred_element_type=jnp.float32)
    t_ref[...] = t
    ps = jnp.sum(t, axis=0, keepdims=True)
    pq = jnp.sum(t * t, axis=0, keepdims=True)

    @pl.when(i == 0)
    def _():
        ssum[...] = ps
        ssq[...] = pq

    @pl.when(i > 0)
    def _():
        ssum[...] += ps
        ssq[...] += pq

    @pl.when(i == NB - 1)
    def _():
        mu = ssum[...] / N
        stats_ref[0:1, :] = mu
        stats_ref[1:2, :] = ssq[...] / N - mu * mu


def _part_specs():
    specs = [pl.BlockSpec((B, F), lambda i: (i, 0))]
    for r in range(R):
        specs.append(pl.BlockSpec((NC, 2, B, FQ),
                                  lambda i, r=r: (0, 0, r * NB + i, 0)))
    for r in range(R):
        specs.append(pl.BlockSpec((NC, B, 8),
                                  lambda i, r=r: (0, r * NB + i, 0)))
    specs += [pl.BlockSpec((F, F), lambda i: (0, 0)),
              pl.BlockSpec((1, F), lambda i: (0, 0)),
              pl.BlockSpec((R, F, F), lambda i: (0, 0, 0))]
    return specs


_tc_accum = pl.pallas_call(
    _tc_accum_body,
    grid=(NB,),
    in_specs=_part_specs(),
    out_specs=[pl.BlockSpec((B, F), lambda i: (i, 0)),
               pl.BlockSpec((8, F), lambda i: (0, 0))],
    out_shape=[jax.ShapeDtypeStruct((N, F), jnp.float32),
               jax.ShapeDtypeStruct((8, F), jnp.float32)],
    scratch_shapes=[pltpu.VMEM((1, F), jnp.float32),
                    pltpu.VMEM((1, F), jnp.float32)],
)


def _normed(t_ref, stats_ref, g_ref, be_ref):
    mu = stats_ref[0:1, :]
    var = stats_ref[1:2, :]
    return ((t_ref[...] - mu) * lax.rsqrt(var + 1e-5) * g_ref[...]
            + be_ref[...])


def _tc_norm_body(t_ref, stats_ref, g_ref, be_ref, hout_ref, hsplit_ref):
    tn = _normed(t_ref, stats_ref, g_ref, be_ref)
    hout_ref[...] = tn
    for q in range(4):
        hsplit_ref[q, :, :] = tn[:, q * FQ:(q + 1) * FQ]


_tc_norm = pl.pallas_call(
    _tc_norm_body,
    grid=(NB,),
    in_specs=[pl.BlockSpec((B, F), lambda i: (i, 0)),
              pl.BlockSpec((8, F), lambda i: (0, 0)),
              pl.BlockSpec((1, F), lambda i: (0, 0)),
              pl.BlockSpec((1, F), lambda i: (0, 0))],
    out_specs=[pl.BlockSpec((B, F), lambda i: (i, 0)),
               pl.BlockSpec((4, B, FQ), lambda i: (0, i, 0))],
    out_shape=[jax.ShapeDtypeStruct((N, F), jnp.float32),
               jax.ShapeDtypeStruct((4, N, FQ), jnp.float32)],
)


def _tc_final_body(t_ref, stats_ref, g_ref, be_ref, batch_ref, l1w_ref,
                   l1b_ref, l2w_ref, l2b_ref, out_ref, seg, cg):
    i = pl.program_id(0)
    tn = _normed(t_ref, stats_ref, g_ref, be_ref)
    gi = lax.broadcasted_iota(jnp.int32, (B, NG), 1)
    oh = (gi == batch_ref[...]).astype(jnp.float32)
    dn = (((0,), (0,)), ((), ()))
    segp = lax.dot_general(oh, tn, dn,
                           preferred_element_type=jnp.float32)
    cgp = lax.dot_general(oh, jnp.ones((B, 8), jnp.float32), dn,
                          preferred_element_type=jnp.float32)

    @pl.when(i == 0)
    def _():
        seg[...] = segp
        cg[...] = cgp

    @pl.when(i > 0)
    def _():
        seg[...] += segp
        cg[...] += cgp

    @pl.when(i == NB - 1)
    def _():
        pooled = seg[...] / jnp.maximum(cg[..., 0:1], 1.0)
        y = jnp.dot(pooled, l1w_ref[...],
                    preferred_element_type=jnp.float32) + l1b_ref[...]
        out_ref[...] = jnp.dot(y, l2w_ref[...],
                               preferred_element_type=jnp.float32) + l2b_ref[...]


_tc_final = pl.pallas_call(
    _tc_final_body,
    grid=(NB,),
    in_specs=[pl.BlockSpec((B, F), lambda i: (i, 0)),
              pl.BlockSpec((8, F), lambda i: (0, 0)),
              pl.BlockSpec((1, F), lambda i: (0, 0)),
              pl.BlockSpec((1, F), lambda i: (0, 0)),
              pl.BlockSpec((B, 1), lambda i: (i, 0)),
              pl.BlockSpec((F, F), lambda i: (0, 0)),
              pl.BlockSpec((1, F), lambda i: (0, 0)),
              pl.BlockSpec((F, F), lambda i: (0, 0)),
              pl.BlockSpec((1, F), lambda i: (0, 0))],
    out_specs=pl.BlockSpec((NG, F), lambda i: (0, 0)),
    out_shape=jax.ShapeDtypeStruct((NG, F), jnp.float32),
    scratch_shapes=[pltpu.VMEM((NG, F), jnp.float32),
                    pltpu.VMEM((NG, 8), jnp.float32)],
)


def _pad_layer(Wl, rootl, bl, gl, bel):
    fi, fo = rootl.shape
    Wl = jnp.pad(Wl, ((0, 0), (0, F - fi), (0, F - fo)))
    rootl = jnp.pad(rootl, ((0, F - fi), (0, F - fo)))
    pad1 = lambda v: jnp.pad(v, (0, F - v.shape[0])).reshape(1, F)
    return Wl, rootl, pad1(bl), pad1(gl), pad1(bel)


def kernel(x, edge_attr, edge_index, edge_type, batch, W0, root0, b0, g0,
           be0, W1, root1, b1, g1, be1, W2, root2, b2, g2, be2, W3, root3,
           b3, g3, be3, lin1_W, lin1_b, lin2_W, lin2_b):
    src = edge_index[0].astype(jnp.int32)
    dst = edge_index[1].astype(jnp.int32)
    et = edge_type.astype(jnp.int32)
    E = src.shape[0]

    src_p = jnp.pad(src, (0, EPAD - E))
    sidx_flat = jnp.pad(et * N + dst, (0, EPAD - E), constant_values=DUMMY)
    gidx16 = src_p.reshape(NT, NSTEP, K)
    gidxa = jnp.concatenate([gidx16[None], gidx16[None] + N], axis=0)
    gidxa = gidxa.reshape(NC * NT, NSTEP, K)
    gidxb = gidxa + 2 * N
    sidx = sidx_flat.reshape(NT, NSTEP, K)
    sidx32 = sidx_flat.reshape(NC * NT, NSTEP32, K)

    zrows = jnp.zeros((RPT, FQ), jnp.float32)
    zrows8 = jnp.zeros((RPT, 8), jnp.float32)
    ones_in = jnp.ones((K, 8), jnp.float32)

    _sc_aggregate, _sc_count = _sc_kernels()
    cpart = _sc_count(sidx32, ones_in, zrows8)

    lw = [_pad_layer(W0, root0, b0, g0, be0),
          _pad_layer(W1, root1, b1, g1, be1),
          _pad_layer(W2, root2, b2, g2, be2),
          _pad_layer(W3, root3, b3, g3, be3)]

    h = x
    hsplit = jnp.concatenate([x[:, q * FQ:(q + 1) * FQ] for q in range(4)],
                             axis=0)
    for l in range(3):
        Wl, rootl, bl, gl, bel = lw[l]
        part = _sc_aggregate(hsplit, gidxa, gidxb, sidx, zrows)
        t, stats = _tc_accum(h, part, part, part, cpart, cpart, cpart,
                             rootl, bl, Wl)
        h, hsplit4 = _tc_norm(t, stats, gl, bel)
        hsplit = hsplit4.reshape(4 * N, FQ)

    Wl, rootl, bl, gl, bel = lw[3]
    part = _sc_aggregate(hsplit, gidxa, gidxb, sidx, zrows)
    t, stats = _tc_accum(h, part, part, part, cpart, cpart, cpart,
                         rootl, bl, Wl)
    l1w = jnp.pad(lin1_W, ((0, 0), (0, F - lin1_W.shape[1])))
    l1b = jnp.pad(lin1_b, (0, F - lin1_b.shape[0])).reshape(1, F)
    l2w = jnp.pad(lin2_W, ((0, F - lin2_W.shape[0]), (0, F - lin2_W.shape[1])))
    l2b = jnp.pad(lin2_b, (0, F - lin2_b.shape[0])).reshape(1, F)
    out = _tc_final(t, stats, gl, bel,
                    batch.astype(jnp.int32).reshape(N, 1), l1w, l1b, l2w,
                    l2b)
    return out[:, 0:1]


# speedup vs baseline: 6.9107x; 1.0195x over previous
import functools

import jax
import jax.numpy as jnp
from jax import lax
from jax.experimental import pallas as pl
from jax.experimental.pallas import tpu as pltpu
from jax.experimental.pallas import tpu_sc as plsc

N = 10000
NG = 64
R = 3
F = 128
FQ = 32
NT = 16
NC = 2
K = 128
NSTEP = 160
EPAD = NT * NSTEP * K
NSTEP32 = EPAD // (NC * NT * K)
NROW = 30080
DUMMY = R * N
RPT = NROW // NT


NBUF = 5


def _sc_aggregate_body(table, gidxa, gidxb, sidx, zrows, out, gv, sv, rows,
                       acc, *sems):
    gsem = sems[:NBUF]
    ssem = sems[NBUF:]
    c = lax.axis_index("c")
    s = lax.axis_index("s")
    w = c * NT + s
    pltpu.sync_copy(sidx.at[s], sv)

    def rnd(j0, carry):
        base = j0 * NBUF
        for b in range(NBUF):
            pltpu.make_async_copy(table.at[gv.at[base + b]], rows.at[b],
                                  gsem[b]).wait()
            pltpu.async_copy(rows.at[b], acc.at[sv.at[base + b]], ssem[b],
                             add=True)
        for b in range(NBUF):
            pltpu.make_async_copy(rows.at[b], acc.at[sv.at[base + b]],
                                  ssem[b]).wait()

            @pl.when(base + NBUF + b < NSTEP)
            def _():
                pltpu.async_copy(table.at[gv.at[base + NBUF + b]],
                                 rows.at[b], gsem[b])
        return carry

    for p, gsrc in enumerate((gidxa, gidxb)):
        pltpu.sync_copy(zrows, acc.at[pl.ds(s * RPT, RPT)])
        pltpu.sync_copy(gsrc.at[w], gv)
        plsc.subcore_barrier()
        for b in range(NBUF):
            pltpu.async_copy(table.at[gv.at[b]], rows.at[b], gsem[b])
        lax.fori_loop(0, NSTEP // NBUF, rnd, 0)
        plsc.subcore_barrier()
        pltpu.sync_copy(acc.at[pl.ds(s * RPT, RPT)],
                        out.at[c].at[p].at[pl.ds(s * RPT, RPT)])


def _sc_count_body(sidx32, ones_in, zrows, out, sv, ones_v, acc):
    c = lax.axis_index("c")
    s = lax.axis_index("s")
    w = c * NT + s
    pltpu.sync_copy(zrows, acc.at[pl.ds(s * RPT, RPT)])
    pltpu.sync_copy(sidx32.at[w], sv)
    pltpu.sync_copy(ones_in, ones_v)
    plsc.subcore_barrier()

    def step(j, carry):
        pltpu.sync_copy(ones_v, acc.at[sv.at[j]], add=True)
        return carry

    lax.fori_loop(0, NSTEP32, step, 0)
    plsc.subcore_barrier()
    pltpu.sync_copy(acc.at[pl.ds(s * RPT, RPT)],
                    out.at[c].at[pl.ds(s * RPT, RPT)])


@functools.cache
def _sc_kernels():
    mesh = plsc.VectorSubcoreMesh(core_axis_name="c", subcore_axis_name="s")
    params = pltpu.CompilerParams(use_tc_tiling_on_sc=False)
    agg = pl.kernel(
        _sc_aggregate_body,
        out_type=jax.ShapeDtypeStruct((NC, 2, NROW, FQ), jnp.float32),
        mesh=mesh,
        compiler_params=params,
        scratch_types=[
            pltpu.VMEM((NSTEP, K), jnp.int32),
            pltpu.VMEM((NSTEP, K), jnp.int32),
            pltpu.VMEM((NBUF, K, FQ), jnp.float32),
            pltpu.VMEM_SHARED((NROW, FQ), jnp.float32),
        ] + [pltpu.SemaphoreType.DMA] * (2 * NBUF),
    )
    cnt = pl.kernel(
        _sc_count_body,
        out_type=jax.ShapeDtypeStruct((NC, NROW, 8), jnp.float32),
        mesh=mesh,
        compiler_params=params,
        scratch_types=[
            pltpu.VMEM((NSTEP32, K), jnp.int32),
            pltpu.VMEM((K, 8), jnp.float32),
            pltpu.VMEM_SHARED((NROW, 8), jnp.float32),
        ],
    )
    return agg, cnt


B = 2000
NB = N // B


def _tc_accum_body(h_ref, p0_ref, p1_ref, p2_ref, c0_ref, c1_ref, c2_ref,
                   root_ref, b_ref, W_ref, t_ref, stats_ref, ssum, ssq):
    i = pl.program_id(0)
    t = jnp.dot(h_ref[...], root_ref[...],
                preferred_element_type=jnp.float32) + b_ref[...]
    for r, (p_ref, c_ref) in enumerate(
            ((p0_ref, c0_ref), (p1_ref, c1_ref), (p2_ref, c2_ref))):
        cnt = c_ref[0, :, 0:1] + c_ref[1, :, 0:1]
        inv = 1.0 / jnp.maximum(cnt, 1.0)
        for q in range(4):
            c, p = q % 2, q // 2
            t = t + jnp.dot(p_ref[c, p, :, :] * inv,
                            W_ref[r, q * FQ:(q + 1) * FQ, :],
                            preferred_element_type=jnp.float32)
    t_ref[...] = t
    ps = jnp.sum(t, axis=0, keepdims=True)
    pq = jnp.sum(t * t, axis=0, keepdims=True)

    @pl.when(i == 0)
    def _():
        ssum[...] = ps
        ssq[...] = pq

    @pl.when(i > 0)
    def _():
        ssum[...] += ps
        ssq[...] += pq

    @pl.when(i == NB - 1)
    def _():
        mu = ssum[...] / N
        stats_ref[0:1, :] = mu
        stats_ref[1:2, :] = ssq[...] / N - mu * mu


def _part_specs():
    specs = [pl.BlockSpec((B, F), lambda i: (i, 0))]
    for r in range(R):
        specs.append(pl.BlockSpec((NC, 2, B, FQ),
                                  lambda i, r=r: (0, 0, r * NB + i, 0)))
    for r in range(R):
        specs.append(pl.BlockSpec((NC, B, 8),
                                  lambda i, r=r: (0, r * NB + i, 0)))
    specs += [pl.BlockSpec((F, F), lambda i: (0, 0)),
              pl.BlockSpec((1, F), lambda i: (0, 0)),
              pl.BlockSpec((R, F, F), lambda i: (0, 0, 0))]
    return specs


_tc_accum = pl.pallas_call(
    _tc_accum_body,
    grid=(NB,),
    in_specs=_part_specs(),
    out_specs=[pl.BlockSpec((B, F), lambda i: (i, 0)),
               pl.BlockSpec((8, F), lambda i: (0, 0))],
    out_shape=[jax.ShapeDtypeStruct((N, F), jnp.float32),
               jax.ShapeDtypeStruct((8, F), jnp.float32)],
    scratch_shapes=[pltpu.VMEM((1, F), jnp.float32),
                    pltpu.VMEM((1, F), jnp.float32)],
)


def _normed(t_ref, stats_ref, g_ref, be_ref):
    mu = stats_ref[0:1, :]
    var = stats_ref[1:2, :]
    return ((t_ref[...] - mu) * lax.rsqrt(var + 1e-5) * g_ref[...]
            + be_ref[...])


def _tc_norm_body(t_ref, stats_ref, g_ref, be_ref, hout_ref, hsplit_ref):
    tn = _normed(t_ref, stats_ref, g_ref, be_ref)
    hout_ref[...] = tn
    for q in range(4):
        hsplit_ref[q, :, :] = tn[:, q * FQ:(q + 1) * FQ]


_tc_norm = pl.pallas_call(
    _tc_norm_body,
    grid=(NB,),
    in_specs=[pl.BlockSpec((B, F), lambda i: (i, 0)),
              pl.BlockSpec((8, F), lambda i: (0, 0)),
              pl.BlockSpec((1, F), lambda i: (0, 0)),
              pl.BlockSpec((1, F), lambda i: (0, 0))],
    out_specs=[pl.BlockSpec((B, F), lambda i: (i, 0)),
               pl.BlockSpec((4, B, FQ), lambda i: (0, i, 0))],
    out_shape=[jax.ShapeDtypeStruct((N, F), jnp.float32),
               jax.ShapeDtypeStruct((4, N, FQ), jnp.float32)],
)


def _tc_final_body(t_ref, stats_ref, g_ref, be_ref, batch_ref, l1w_ref,
                   l1b_ref, l2w_ref, l2b_ref, out_ref, seg, cg):
    i = pl.program_id(0)
    tn = _normed(t_ref, stats_ref, g_ref, be_ref)
    gi = lax.broadcasted_iota(jnp.int32, (B, NG), 1)
    oh = (gi == batch_ref[...]).astype(jnp.float32)
    dn = (((0,), (0,)), ((), ()))
    segp = lax.dot_general(oh, tn, dn,
                           preferred_element_type=jnp.float32)
    cgp = lax.dot_general(oh, jnp.ones((B, 8), jnp.float32), dn,
                          preferred_element_type=jnp.float32)

    @pl.when(i == 0)
    def _():
        seg[...] = segp
        cg[...] = cgp

    @pl.when(i > 0)
    def _():
        seg[...] += segp
        cg[...] += cgp

    @pl.when(i == NB - 1)
    def _():
        pooled = seg[...] / jnp.maximum(cg[..., 0:1], 1.0)
        y = jnp.dot(pooled, l1w_ref[...],
                    preferred_element_type=jnp.float32) + l1b_ref[...]
        out_ref[...] = jnp.dot(y, l2w_ref[...],
                               preferred_element_type=jnp.float32) + l2b_ref[...]


_tc_final = pl.pallas_call(
    _tc_final_body,
    grid=(NB,),
    in_specs=[pl.BlockSpec((B, F), lambda i: (i, 0)),
              pl.BlockSpec((8, F), lambda i: (0, 0)),
              pl.BlockSpec((1, F), lambda i: (0, 0)),
              pl.BlockSpec((1, F), lambda i: (0, 0)),
              pl.BlockSpec((B, 1), lambda i: (i, 0)),
              pl.BlockSpec((F, F), lambda i: (0, 0)),
              pl.BlockSpec((1, F), lambda i: (0, 0)),
              pl.BlockSpec((F, F), lambda i: (0, 0)),
              pl.BlockSpec((1, F), lambda i: (0, 0))],
    out_specs=pl.BlockSpec((NG, F), lambda i: (0, 0)),
    out_shape=jax.ShapeDtypeStruct((NG, F), jnp.float32),
    scratch_shapes=[pltpu.VMEM((NG, F), jnp.float32),
                    pltpu.VMEM((NG, 8), jnp.float32)],
)


def _pad_layer(Wl, rootl, bl, gl, bel):
    fi, fo = rootl.shape
    Wl = jnp.pad(Wl, ((0, 0), (0, F - fi), (0, F - fo)))
    rootl = jnp.pad(rootl, ((0, F - fi), (0, F - fo)))
    pad1 = lambda v: jnp.pad(v, (0, F - v.shape[0])).reshape(1, F)
    return Wl, rootl, pad1(bl), pad1(gl), pad1(bel)


def kernel(x, edge_attr, edge_index, edge_type, batch, W0, root0, b0, g0,
           be0, W1, root1, b1, g1, be1, W2, root2, b2, g2, be2, W3, root3,
           b3, g3, be3, lin1_W, lin1_b, lin2_W, lin2_b):
    src = edge_index[0].astype(jnp.int32)
    dst = edge_index[1].astype(jnp.int32)
    et = edge_type.astype(jnp.int32)
    E = src.shape[0]

    src_p = jnp.pad(src, (0, EPAD - E))
    sidx_flat = jnp.pad(et * N + dst, (0, EPAD - E), constant_values=DUMMY)
    gidx16 = src_p.reshape(NT, NSTEP, K)
    gidxa = jnp.concatenate([gidx16[None], gidx16[None] + N], axis=0)
    gidxa = gidxa.reshape(NC * NT, NSTEP, K)
    gidxb = gidxa + 2 * N
    sidx = sidx_flat.reshape(NT, NSTEP, K)
    sidx32 = sidx_flat.reshape(NC * NT, NSTEP32, K)

    zrows = jnp.zeros((RPT, FQ), jnp.float32)
    zrows8 = jnp.zeros((RPT, 8), jnp.float32)
    ones_in = jnp.ones((K, 8), jnp.float32)

    _sc_aggregate, _sc_count = _sc_kernels()
    cpart = _sc_count(sidx32, ones_in, zrows8)

    lw = [_pad_layer(W0, root0, b0, g0, be0),
          _pad_layer(W1, root1, b1, g1, be1),
          _pad_layer(W2, root2, b2, g2, be2),
          _pad_layer(W3, root3, b3, g3, be3)]

    h = x
    hsplit = jnp.concatenate([x[:, q * FQ:(q + 1) * FQ] for q in range(4)],
                             axis=0)
    for l in range(3):
        Wl, rootl, bl, gl, bel = lw[l]
        part = _sc_aggregate(hsplit, gidxa, gidxb, sidx, zrows)
        t, stats = _tc_accum(h, part, part, part, cpart, cpart, cpart,
                             rootl, bl, Wl)
        h, hsplit4 = _tc_norm(t, stats, gl, bel)
        hsplit = hsplit4.reshape(4 * N, FQ)

    Wl, rootl, bl, gl, bel = lw[3]
    part = _sc_aggregate(hsplit, gidxa, gidxb, sidx, zrows)
    t, stats = _tc_accum(h, part, part, part, cpart, cpart, cpart,
                         rootl, bl, Wl)
    l1w = jnp.pad(lin1_W, ((0, 0), (0, F - lin1_W.shape[1])))
    l1b = jnp.pad(lin1_b, (0, F - lin1_b.shape[0])).reshape(1, F)
    l2w = jnp.pad(lin2_W, ((0, F - lin2_W.shape[0]), (0, F - lin2_W.shape[1])))
    l2b = jnp.pad(lin2_b, (0, F - lin2_b.shape[0])).reshape(1, F)
    out = _tc_final(t, stats, gl, bel,
                    batch.astype(jnp.int32).reshape(N, 1), l1w, l1b, l2w,
                    l2b)
    return out[:, 0:1]


# speedup vs baseline: 7.4747x; 1.0816x over previous
import functools

import jax
import jax.numpy as jnp
from jax import lax
from jax.experimental import pallas as pl
from jax.experimental.pallas import tpu as pltpu
from jax.experimental.pallas import tpu_sc as plsc

N = 10000
NG = 64
R = 3
F = 128
FQ = 32
NT = 16
NC = 2
K = 128
NSTEP = 160
EPAD = NT * NSTEP * K
NSTEP32 = EPAD // (NC * NT * K)
NROW = 30080
DUMMY = R * N
RPT = NROW // NT


NBUF = 5


def _sc_aggregate_body(table, gidxa, gidxb, sidx, zrows, out, gv, sv, rows,
                       acc, *sems):
    gsem = sems[:NBUF]
    ssem = sems[NBUF:]
    c = lax.axis_index("c")
    s = lax.axis_index("s")
    w = c * NT + s
    pltpu.sync_copy(sidx.at[s], sv)

    def rnd(j0, carry):
        base = j0 * NBUF
        for b in range(NBUF):
            pltpu.make_async_copy(table.at[gv.at[base + b]], rows.at[b],
                                  gsem[b]).wait()
            pltpu.async_copy(rows.at[b], acc.at[sv.at[base + b]], ssem[b],
                             add=True)
        for b in range(NBUF):
            pltpu.make_async_copy(rows.at[b], acc.at[sv.at[base + b]],
                                  ssem[b]).wait()

            @pl.when(base + NBUF + b < NSTEP)
            def _():
                pltpu.async_copy(table.at[gv.at[base + NBUF + b]],
                                 rows.at[b], gsem[b])
        return carry

    for p, gsrc in enumerate((gidxa, gidxb)):
        pltpu.sync_copy(zrows, acc.at[pl.ds(s * RPT, RPT)])
        pltpu.sync_copy(gsrc.at[w], gv)
        plsc.subcore_barrier()
        for b in range(NBUF):
            pltpu.async_copy(table.at[gv.at[b]], rows.at[b], gsem[b])
        lax.fori_loop(0, NSTEP // NBUF, rnd, 0)
        plsc.subcore_barrier()
        pltpu.sync_copy(acc.at[pl.ds(s * RPT, RPT)],
                        out.at[c].at[p].at[pl.ds(s * RPT, RPT)])


def _sc_count_body(sidx32, ones_in, zrows, out, sv, ones_v, acc):
    c = lax.axis_index("c")
    s = lax.axis_index("s")
    w = c * NT + s
    pltpu.sync_copy(zrows, acc.at[pl.ds(s * RPT, RPT)])
    pltpu.sync_copy(sidx32.at[w], sv)
    pltpu.sync_copy(ones_in, ones_v)
    plsc.subcore_barrier()

    def step(j, carry):
        pltpu.sync_copy(ones_v, acc.at[sv.at[j]], add=True)
        return carry

    lax.fori_loop(0, NSTEP32, step, 0)
    plsc.subcore_barrier()
    pltpu.sync_copy(acc.at[pl.ds(s * RPT, RPT)],
                    out.at[c].at[pl.ds(s * RPT, RPT)])


@functools.cache
def _sc_kernels():
    mesh = plsc.VectorSubcoreMesh(core_axis_name="c", subcore_axis_name="s")
    params = pltpu.CompilerParams(use_tc_tiling_on_sc=False)
    agg = pl.kernel(
        _sc_aggregate_body,
        out_type=jax.ShapeDtypeStruct((NC, 2, NROW, FQ), jnp.float32),
        mesh=mesh,
        compiler_params=params,
        scratch_types=[
            pltpu.VMEM((NSTEP, K), jnp.int32),
            pltpu.VMEM((NSTEP, K), jnp.int32),
            pltpu.VMEM((NBUF, K, FQ), jnp.float32),
            pltpu.VMEM_SHARED((NROW, FQ), jnp.float32),
        ] + [pltpu.SemaphoreType.DMA] * (2 * NBUF),
    )
    cnt = pl.kernel(
        _sc_count_body,
        out_type=jax.ShapeDtypeStruct((NC, NROW, 8), jnp.float32),
        mesh=mesh,
        compiler_params=params,
        scratch_types=[
            pltpu.VMEM((NSTEP32, K), jnp.int32),
            pltpu.VMEM((K, 8), jnp.float32),
            pltpu.VMEM_SHARED((NROW, 8), jnp.float32),
        ],
    )
    return agg, cnt


B = 2000
NB = N // B


def _tc_accum_body(h_ref, p0_ref, p1_ref, p2_ref, c0_ref, c1_ref, c2_ref,
                   statsp_ref, gp_ref, bep_ref, root_ref, b_ref, W_ref,
                   t_ref, tsplit_ref, stats_ref, ssum, ssq):
    i = pl.program_id(0)
    mu = statsp_ref[0:1, :]
    var = statsp_ref[1:2, :]
    a = lax.rsqrt(var + 1e-5) * gp_ref[...]
    d = bep_ref[...] - mu * a
    hn = h_ref[...] * a + d
    t = jnp.dot(hn, root_ref[...],
                preferred_element_type=jnp.float32) + b_ref[...]
    for r, (p_ref, c_ref) in enumerate(
            ((p0_ref, c0_ref), (p1_ref, c1_ref), (p2_ref, c2_ref))):
        cnt = c_ref[0, :, 0:1] + c_ref[1, :, 0:1]
        inv = 1.0 / jnp.maximum(cnt, 1.0)
        has = jnp.minimum(cnt, 1.0)
        for q in range(4):
            c, p = q % 2, q // 2
            aq = a[0:1, q * FQ:(q + 1) * FQ]
            t = t + jnp.dot(p_ref[c, p, :, :] * inv * aq,
                            W_ref[r, q * FQ:(q + 1) * FQ, :],
                            preferred_element_type=jnp.float32)
        dW = jnp.dot(d, W_ref[r], preferred_element_type=jnp.float32)
        t = t + has * dW
    t_ref[...] = t
    for q in range(4):
        tsplit_ref[q, :, :] = t[:, q * FQ:(q + 1) * FQ]
    ps = jnp.sum(t, axis=0, keepdims=True)
    pq = jnp.sum(t * t, axis=0, keepdims=True)

    @pl.when(i == 0)
    def _():
        ssum[...] = ps
        ssq[...] = pq

    @pl.when(i > 0)
    def _():
        ssum[...] += ps
        ssq[...] += pq

    @pl.when(i == NB - 1)
    def _():
        mu = ssum[...] / N
        stats_ref[0:1, :] = mu
        stats_ref[1:2, :] = ssq[...] / N - mu * mu


def _part_specs():
    specs = [pl.BlockSpec((B, F), lambda i: (i, 0))]
    for r in range(R):
        specs.append(pl.BlockSpec((NC, 2, B, FQ),
                                  lambda i, r=r: (0, 0, r * NB + i, 0)))
    for r in range(R):
        specs.append(pl.BlockSpec((NC, B, 8),
                                  lambda i, r=r: (0, r * NB + i, 0)))
    specs += [pl.BlockSpec((8, F), lambda i: (0, 0)),
              pl.BlockSpec((1, F), lambda i: (0, 0)),
              pl.BlockSpec((1, F), lambda i: (0, 0)),
              pl.BlockSpec((F, F), lambda i: (0, 0)),
              pl.BlockSpec((1, F), lambda i: (0, 0)),
              pl.BlockSpec((R, F, F), lambda i: (0, 0, 0))]
    return specs


_tc_accum = pl.pallas_call(
    _tc_accum_body,
    grid=(NB,),
    in_specs=_part_specs(),
    out_specs=[pl.BlockSpec((B, F), lambda i: (i, 0)),
               pl.BlockSpec((4, B, FQ), lambda i: (0, i, 0)),
               pl.BlockSpec((8, F), lambda i: (0, 0))],
    out_shape=[jax.ShapeDtypeStruct((N, F), jnp.float32),
               jax.ShapeDtypeStruct((4, N, FQ), jnp.float32),
               jax.ShapeDtypeStruct((8, F), jnp.float32)],
    scratch_shapes=[pltpu.VMEM((1, F), jnp.float32),
                    pltpu.VMEM((1, F), jnp.float32)],
)


def _normed(t_ref, stats_ref, g_ref, be_ref):
    mu = stats_ref[0:1, :]
    var = stats_ref[1:2, :]
    return ((t_ref[...] - mu) * lax.rsqrt(var + 1e-5) * g_ref[...]
            + be_ref[...])


def _tc_final_body(t_ref, stats_ref, g_ref, be_ref, batch_ref, l1w_ref,
                   l1b_ref, l2w_ref, l2b_ref, out_ref, seg, cg):
    i = pl.program_id(0)
    tn = _normed(t_ref, stats_ref, g_ref, be_ref)
    gi = lax.broadcasted_iota(jnp.int32, (B, NG), 1)
    oh = (gi == batch_ref[...]).astype(jnp.float32)
    dn = (((0,), (0,)), ((), ()))
    segp = lax.dot_general(oh, tn, dn,
                           preferred_element_type=jnp.float32)
    cgp = lax.dot_general(oh, jnp.ones((B, 8), jnp.float32), dn,
                          preferred_element_type=jnp.float32)

    @pl.when(i == 0)
    def _():
        seg[...] = segp
        cg[...] = cgp

    @pl.when(i > 0)
    def _():
        seg[...] += segp
        cg[...] += cgp

    @pl.when(i == NB - 1)
    def _():
        pooled = seg[...] / jnp.maximum(cg[..., 0:1], 1.0)
        y = jnp.dot(pooled, l1w_ref[...],
                    preferred_element_type=jnp.float32) + l1b_ref[...]
        out_ref[...] = jnp.dot(y, l2w_ref[...],
                               preferred_element_type=jnp.float32) + l2b_ref[...]


_tc_final = pl.pallas_call(
    _tc_final_body,
    grid=(NB,),
    in_specs=[pl.BlockSpec((B, F), lambda i: (i, 0)),
              pl.BlockSpec((8, F), lambda i: (0, 0)),
              pl.BlockSpec((1, F), lambda i: (0, 0)),
              pl.BlockSpec((1, F), lambda i: (0, 0)),
              pl.BlockSpec((B, 1), lambda i: (i, 0)),
              pl.BlockSpec((F, F), lambda i: (0, 0)),
              pl.BlockSpec((1, F), lambda i: (0, 0)),
              pl.BlockSpec((F, F), lambda i: (0, 0)),
              pl.BlockSpec((1, F), lambda i: (0, 0))],
    out_specs=pl.BlockSpec((NG, F), lambda i: (0, 0)),
    out_shape=jax.ShapeDtypeStruct((NG, F), jnp.float32),
    scratch_shapes=[pltpu.VMEM((NG, F), jnp.float32),
                    pltpu.VMEM((NG, 8), jnp.float32)],
)


def _pad_layer(Wl, rootl, bl, gl, bel):
    fi, fo = rootl.shape
    Wl = jnp.pad(Wl, ((0, 0), (0, F - fi), (0, F - fo)))
    rootl = jnp.pad(rootl, ((0, F - fi), (0, F - fo)))
    pad1 = lambda v: jnp.pad(v, (0, F - v.shape[0])).reshape(1, F)
    return Wl, rootl, pad1(bl), pad1(gl), pad1(bel)


def kernel(x, edge_attr, edge_index, edge_type, batch, W0, root0, b0, g0,
           be0, W1, root1, b1, g1, be1, W2, root2, b2, g2, be2, W3, root3,
           b3, g3, be3, lin1_W, lin1_b, lin2_W, lin2_b):
    src = edge_index[0].astype(jnp.int32)
    dst = edge_index[1].astype(jnp.int32)
    et = edge_type.astype(jnp.int32)
    E = src.shape[0]

    src_p = jnp.pad(src, (0, EPAD - E))
    sidx_flat = jnp.pad(et * N + dst, (0, EPAD - E), constant_values=DUMMY)
    gidx16 = src_p.reshape(NT, NSTEP, K)
    gidxa = jnp.concatenate([gidx16[None], gidx16[None] + N], axis=0)
    gidxa = gidxa.reshape(NC * NT, NSTEP, K)
    gidxb = gidxa + 2 * N
    sidx = sidx_flat.reshape(NT, NSTEP, K)
    sidx32 = sidx_flat.reshape(NC * NT, NSTEP32, K)

    zrows = jnp.zeros((RPT, FQ), jnp.float32)
    zrows8 = jnp.zeros((RPT, 8), jnp.float32)
    ones_in = jnp.ones((K, 8), jnp.float32)

    _sc_aggregate, _sc_count = _sc_kernels()
    cpart = _sc_count(sidx32, ones_in, zrows8)

    lw = [_pad_layer(W0, root0, b0, g0, be0),
          _pad_layer(W1, root1, b1, g1, be1),
          _pad_layer(W2, root2, b2, g2, be2),
          _pad_layer(W3, root3, b3, g3, be3)]

    stats_c = jnp.concatenate(
        [jnp.zeros((1, F), jnp.float32),
         jnp.full((1, F), 1.0 - 1e-5, jnp.float32),
         jnp.zeros((6, F), jnp.float32)], axis=0)
    g_c = jnp.ones((1, F), jnp.float32)
    be_c = jnp.zeros((1, F), jnp.float32)

    t = x
    hsplit = jnp.concatenate([x[:, q * FQ:(q + 1) * FQ] for q in range(4)],
                             axis=0)
    for l in range(4):
        Wl, rootl, bl, gl, bel = lw[l]
        part = _sc_aggregate(hsplit, gidxa, gidxb, sidx, zrows)
        t, tsplit4, stats = _tc_accum(t, part, part, part, cpart, cpart,
                                      cpart, stats_c, g_c, be_c, rootl, bl,
                                      Wl)
        hsplit = tsplit4.reshape(4 * N, FQ)
        stats_c, g_c, be_c = stats, gl, bel
    l1w = jnp.pad(lin1_W, ((0, 0), (0, F - lin1_W.shape[1])))
    l1b = jnp.pad(lin1_b, (0, F - lin1_b.shape[0])).reshape(1, F)
    l2w = jnp.pad(lin2_W, ((0, F - lin2_W.shape[0]), (0, F - lin2_W.shape[1])))
    l2b = jnp.pad(lin2_b, (0, F - lin2_b.shape[0])).reshape(1, F)
    out = _tc_final(t, stats, gl, bel,
                    batch.astype(jnp.int32).reshape(N, 1), l1w, l1b, l2w,
                    l2b)
    return out[:, 0:1]
